# initial kernel scaffold (unmeasured)
import jax
import jax.numpy as jnp
from jax import lax
from jax.experimental import pallas as pl
from jax.experimental.pallas import tpu as pltpu


def kernel(
    x,
):
    def body(*refs):
        pass

    out_shape = jax.ShapeDtypeStruct(..., jnp.float32)
    return pl.pallas_call(body, out_shape=out_shape)(...)


# baseline (device time: 53023 ns/iter reference)
import jax
import jax.numpy as jnp
from jax import lax
from jax.experimental import pallas as pl
from jax.experimental.pallas import tpu as pltpu


def kernel(x):
    m, n = x.shape

    def body(x_ref, out_ref, comm_ref, send_sem, recv_sem):
        my_x = lax.axis_index("x")
        my_y = lax.axis_index("y")
        my_z = lax.axis_index("z")
        partner = (1 - my_x, my_y, my_z)

        barrier = pltpu.get_barrier_semaphore()
        pl.semaphore_signal(
            barrier, inc=1, device_id=partner,
            device_id_type=pl.DeviceIdType.MESH,
        )
        pl.semaphore_wait(barrier, 1)

        rdma = pltpu.make_async_remote_copy(
            src_ref=x_ref,
            dst_ref=comm_ref,
            send_sem=send_sem,
            recv_sem=recv_sem,
            device_id=partner,
            device_id_type=pl.DeviceIdType.MESH,
        )
        rdma.start()
        rdma.wait()

        out_ref[...] = x_ref[...] + comm_ref[...]

    return pl.pallas_call(
        body,
        out_shape=jax.ShapeDtypeStruct((m, n), x.dtype),
        in_specs=[pl.BlockSpec(memory_space=pltpu.VMEM)],
        out_specs=pl.BlockSpec(memory_space=pltpu.VMEM),
        scratch_shapes=[
            pltpu.VMEM((m, n), x.dtype),
            pltpu.SemaphoreType.DMA,
            pltpu.SemaphoreType.DMA,
        ],
        compiler_params=pltpu.CompilerParams(collective_id=0),
    )(x)


# device time: 35607 ns/iter; 1.4891x vs baseline; 1.4891x over previous
import jax
import jax.numpy as jnp
from jax import lax
from jax.experimental import pallas as pl
from jax.experimental.pallas import tpu as pltpu

N_CHUNKS = 8


def kernel(x):
    m, n = x.shape
    half = m // 2
    ck = half // N_CHUNKS

    def body(x_ref, out_ref, xcomm_ref, x_send_sems, x_recv_sems,
             y_send_sems, y_recv_sems):
        my_x = lax.axis_index("x")
        my_y = lax.axis_index("y")
        my_z = lax.axis_index("z")
        x_partner = (1 - my_x, my_y, my_z)
        y_partner = (my_x, 1 - my_y, my_z)

        half_off = my_y * half
        other_off = (1 - my_y) * half

        barrier = pltpu.get_barrier_semaphore()
        pl.semaphore_signal(
            barrier, inc=1, device_id=x_partner,
            device_id_type=pl.DeviceIdType.MESH,
        )
        pl.semaphore_signal(
            barrier, inc=1, device_id=y_partner,
            device_id_type=pl.DeviceIdType.MESH,
        )
        pl.semaphore_wait(barrier, 2)

        x_rdmas = []
        for c in range(N_CHUNKS):
            r = pltpu.make_async_remote_copy(
                src_ref=x_ref.at[pl.ds(half_off + c * ck, ck), :],
                dst_ref=xcomm_ref.at[c],
                send_sem=x_send_sems.at[c],
                recv_sem=x_recv_sems.at[c],
                device_id=x_partner,
                device_id_type=pl.DeviceIdType.MESH,
            )
            r.start()
            x_rdmas.append(r)

        y_rdmas = []
        for c in range(N_CHUNKS):
            x_rdmas[c].wait_recv()
            rows = pl.ds(half_off + c * ck, ck)
            out_ref[rows, :] = x_ref[rows, :] + xcomm_ref[c, :, :]
            ry = pltpu.make_async_remote_copy(
                src_ref=out_ref.at[rows, :],
                dst_ref=out_ref.at[rows, :],
                send_sem=y_send_sems.at[c],
                recv_sem=y_recv_sems.at[c],
                device_id=y_partner,
                device_id_type=pl.DeviceIdType.MESH,
            )
            ry.start()
            y_rdmas.append(ry)

        for c in range(N_CHUNKS):
            recv = pltpu.make_async_remote_copy(
                src_ref=out_ref.at[pl.ds(half_off + c * ck, ck), :],
                dst_ref=out_ref.at[pl.ds(other_off + c * ck, ck), :],
                send_sem=y_send_sems.at[c],
                recv_sem=y_recv_sems.at[c],
                device_id=y_partner,
                device_id_type=pl.DeviceIdType.MESH,
            )
            recv.wait_recv()
        for c in range(N_CHUNKS):
            x_rdmas[c].wait_send()
            y_rdmas[c].wait_send()

    return pl.pallas_call(
        body,
        out_shape=jax.ShapeDtypeStruct((m, n), x.dtype),
        in_specs=[pl.BlockSpec(memory_space=pltpu.VMEM)],
        out_specs=pl.BlockSpec(memory_space=pltpu.VMEM),
        scratch_shapes=[
            pltpu.VMEM((N_CHUNKS, ck, n), x.dtype),
            pltpu.SemaphoreType.DMA((N_CHUNKS,)),
            pltpu.SemaphoreType.DMA((N_CHUNKS,)),
            pltpu.SemaphoreType.DMA((N_CHUNKS,)),
            pltpu.SemaphoreType.DMA((N_CHUNKS,)),
        ],
        compiler_params=pltpu.CompilerParams(collective_id=0),
    )(x)


# device time: 34259 ns/iter; 1.5477x vs baseline; 1.0393x over previous
import jax
import jax.numpy as jnp
from jax import lax
from jax.experimental import pallas as pl
from jax.experimental.pallas import tpu as pltpu

N_CHUNKS = 32


def kernel(x):
    m, n = x.shape
    half = m // 2
    ck = half // N_CHUNKS

    def body(x_ref, out_ref, xcomm_ref, ycomm_ref,
             x_send_sems, x_recv_sems, y_send_sems, y_recv_sems):
        my_x = lax.axis_index("x")
        my_y = lax.axis_index("y")
        my_z = lax.axis_index("z")
        x_partner = (1 - my_x, my_y, my_z)
        y_partner = (my_x, 1 - my_y, my_z)

        half_off = my_y * half
        other_off = (1 - my_y) * half

        barrier = pltpu.get_barrier_semaphore()
        pl.semaphore_signal(
            barrier, inc=1, device_id=x_partner,
            device_id_type=pl.DeviceIdType.MESH,
        )
        pl.semaphore_signal(
            barrier, inc=1, device_id=y_partner,
            device_id_type=pl.DeviceIdType.MESH,
        )
        pl.semaphore_wait(barrier, 2)

        x_rdmas = []
        for c in range(N_CHUNKS):
            r = pltpu.make_async_remote_copy(
                src_ref=x_ref.at[pl.ds(half_off + c * ck, ck), :],
                dst_ref=xcomm_ref.at[c],
                send_sem=x_send_sems.at[c],
                recv_sem=x_recv_sems.at[c],
                device_id=x_partner,
                device_id_type=pl.DeviceIdType.MESH,
            )
            r.start()
            x_rdmas.append(r)

        y_rdmas = []
        for c in range(N_CHUNKS):
            x_rdmas[c].wait_recv()
            ry = pltpu.make_async_remote_copy(
                src_ref=xcomm_ref.at[c],
                dst_ref=ycomm_ref.at[c],
                send_sem=y_send_sems.at[c],
                recv_sem=y_recv_sems.at[c],
                device_id=y_partner,
                device_id_type=pl.DeviceIdType.MESH,
            )
            ry.start()
            y_rdmas.append(ry)
            rows = pl.ds(half_off + c * ck, ck)
            out_ref[rows, :] = x_ref[rows, :] + xcomm_ref[c, :, :]

        for c in range(N_CHUNKS):
            y_rdmas[c].wait_recv()
            rows = pl.ds(other_off + c * ck, ck)
            out_ref[rows, :] = x_ref[rows, :] + ycomm_ref[c, :, :]

        for c in range(N_CHUNKS):
            x_rdmas[c].wait_send()
            y_rdmas[c].wait_send()

    return pl.pallas_call(
        body,
        out_shape=jax.ShapeDtypeStruct((m, n), x.dtype),
        in_specs=[pl.BlockSpec(memory_space=pltpu.VMEM)],
        out_specs=pl.BlockSpec(memory_space=pltpu.VMEM),
        scratch_shapes=[
            pltpu.VMEM((N_CHUNKS, ck, n), x.dtype),
            pltpu.VMEM((N_CHUNKS, ck, n), x.dtype),
            pltpu.SemaphoreType.DMA((N_CHUNKS,)),
            pltpu.SemaphoreType.DMA((N_CHUNKS,)),
            pltpu.SemaphoreType.DMA((N_CHUNKS,)),
            pltpu.SemaphoreType.DMA((N_CHUNKS,)),
        ],
        compiler_params=pltpu.CompilerParams(collective_id=0),
    )(x)
